# baseline (speedup 1.0000x reference)
import jax
import jax.numpy as jnp
from jax import lax
from jax.experimental import pallas as pl
from jax.experimental.pallas import tpu as pltpu
from jax.experimental.pallas import tpu_sc as plsc

N = 10000
E = 320000
D = 128
NC = 2
NS = 16
NW = NC * NS
L = 16

IPW = (2 * E) // NW

CHUNK = 128
NCHUNK = 160
GCHUNK = NW * NCHUNK
NPAD = 10240
DUMMY_DST = N + 100
ROWS_PER_TILE = NPAD // NS
EXP_CHUNK = 64
NEXP = ROWS_PER_TILE // EXP_CHUNK

def _mesh():
    return plsc.VectorSubcoreMesh(core_axis_name="c", subcore_axis_name="s")


def _deg_body(ei_hbm, degs_hbm, idx_v, hist_v):
    c = lax.axis_index("c")
    s = lax.axis_index("s")
    wid = c * NS + s

    zeros = jnp.zeros((L,), jnp.float32)

    def zero(i, carry):
        hist_v[pl.ds(i * L, L)] = zeros
        return carry

    lax.fori_loop(0, N // L, zero, 0)

    pltpu.sync_copy(ei_hbm.at[pl.ds(wid * IPW, IPW)], idx_v)

    ones = jnp.ones((L,), jnp.float32)

    def accum(i, carry):
        idx = idx_v[pl.ds(i * L, L)]
        plsc.addupdate_scatter(hist_v, [idx], ones)
        return carry

    lax.fori_loop(0, IPW // L, accum, 0)

    pltpu.sync_copy(hist_v, degs_hbm.at[wid])


def _deg_call(ei_flat):
    fn = pl.kernel(
        _deg_body,
        out_type=jax.ShapeDtypeStruct((NW, N), jnp.float32),
        mesh=_mesh(),
        scratch_types=[
            pltpu.VMEM((IPW,), jnp.int32),
            pltpu.VMEM((N,), jnp.float32),
        ],
        compiler_params=pltpu.CompilerParams(needs_layout_passes=False),
    )
    return fn(ei_flat)


def _mm_body(x_ref, wt_ref, b_ref, degs_ref, hn_ref):
    d = jnp.sum(degs_ref[...], axis=0) + 1.0
    dinv = lax.rsqrt(d)
    h = jnp.dot(x_ref[...], wt_ref[...], preferred_element_type=jnp.float32)
    hn_ref[...] = (h + b_ref[...]) * dinv[:, None]


def _mm_call(x, wt, b2, degs):
    return pl.pallas_call(
        _mm_body,
        out_shape=jax.ShapeDtypeStruct((N, D), jnp.float32),
    )(x, wt, b2, degs)


def _agg_body(hn_hbm, cidx_hbm, out_hbm,
              cs0, cs1, ds0, ds1, buf0, buf1, stage_v,
              acc_sh,
              isem0, isem1, gsem0, gsem1, ssem0, ssem1):
    c = lax.axis_index("c")
    s = lax.axis_index("s")
    wid = c * NS + s

    zeros = jnp.zeros((L,), jnp.float32)

    def zero(i, carry):
        r = lax.shift_right_logical(i, 3)
        col = lax.bitwise_and(i, 7)
        stage_v[r, pl.ds(col * L, L)] = zeros
        return carry

    lax.fori_loop(0, EXP_CHUNK * (D // L), zero, 0)

    for j in range(0):
        base = s * ROWS_PER_TILE + j * EXP_CHUNK
        pltpu.sync_copy(stage_v, acc_sh.at[pl.ds(base, EXP_CHUNK)])
    plsc.subcore_barrier()

    slots = ((cs0, ds0, buf0, isem0, gsem0, ssem0),
             (cs1, ds1, buf1, isem1, gsem1, ssem1))

    def chunk_of(k):
        return k * NW + wid

    def copy_ds(p):
        cs, ds = slots[p][0], slots[p][1]
        for kk in range(CHUNK // L):
            ds[pl.ds(kk * L, L)] = cs[1, pl.ds(kk * L, L)]

    def start_idx(k, p):
        pltpu.async_copy(cidx_hbm.at[chunk_of(k)], slots[p][0], slots[p][3])

    def wait_idx(k, p):
        pltpu.make_async_copy(cidx_hbm.at[chunk_of(k)], slots[p][0],
                              slots[p][3]).wait()

    def start_gather(p):
        pltpu.async_copy(hn_hbm.at[slots[p][0].at[0]], slots[p][2],
                         slots[p][4])

    def wait_gather(p):
        pltpu.make_async_copy(hn_hbm.at[slots[p][0].at[0]], slots[p][2],
                              slots[p][4]).wait()

    def start_scatter(p):
        pltpu.async_copy(slots[p][2], acc_sh.at[slots[p][1]], slots[p][5],
                         add=True)

    def wait_scatter(p):
        pltpu.make_async_copy(slots[p][2], acc_sh.at[slots[p][1]],
                              slots[p][5]).wait()

    def step(k, p, do_idx=True, do_gather=True, do_wait_scatter=True):
        q = 1 - p
        if do_gather:
            wait_idx(k + 1, q)
            if False and do_wait_scatter:
                wait_scatter(q)
            start_gather(q)
        wait_gather(p)
        copy_ds(p)
        if False:
            start_scatter(p)
        if do_idx:
            start_idx(k + 2, p)

    start_idx(jnp.int32(0), 0)
    wait_idx(jnp.int32(0), 0)
    start_gather(0)
    start_idx(jnp.int32(1), 1)

    step(jnp.int32(0), 0, do_wait_scatter=False)

    def pair(t, carry):
        step(2 * t + 1, 1)
        step(2 * t + 2, 0)
        return carry

    lax.fori_loop(0, (NCHUNK - 4) // 2, pair, 0)

    step(jnp.int32(NCHUNK - 3), 1)
    step(jnp.int32(NCHUNK - 2), 0, do_idx=False)
    step(jnp.int32(NCHUNK - 1), 1, do_idx=False, do_gather=False)
    if False:
        wait_scatter(0)
        wait_scatter(1)

    plsc.subcore_barrier()

    for j in range(0):
        base = s * ROWS_PER_TILE + j * EXP_CHUNK
        pltpu.sync_copy(acc_sh.at[pl.ds(base, EXP_CHUNK)], stage_v)
        pltpu.sync_copy(stage_v, out_hbm.at[c, pl.ds(base, EXP_CHUNK)])


def _agg_call(hn, cidx):
    fn = pl.kernel(
        _agg_body,
        out_type=jax.ShapeDtypeStruct((NC, NPAD, D), jnp.float32),
        mesh=_mesh(),
        scratch_types=[
            pltpu.VMEM((2, CHUNK), jnp.int32),
            pltpu.VMEM((2, CHUNK), jnp.int32),
            pltpu.VMEM((CHUNK,), jnp.int32),
            pltpu.VMEM((CHUNK,), jnp.int32),
            pltpu.VMEM((CHUNK, 2 * D), jnp.float32),
            pltpu.VMEM((CHUNK, 2 * D), jnp.float32),
            pltpu.VMEM((EXP_CHUNK, D), jnp.float32),
            pltpu.VMEM_SHARED((128, D), jnp.float32),
            pltpu.SemaphoreType.DMA,
            pltpu.SemaphoreType.DMA,
            pltpu.SemaphoreType.DMA,
            pltpu.SemaphoreType.DMA,
            pltpu.SemaphoreType.DMA,
            pltpu.SemaphoreType.DMA,
        ],
        compiler_params=pltpu.CompilerParams(needs_layout_passes=False),
    )
    return fn(hn, cidx)


def _comb_body(acc_ref, hn_ref, degs_ref, o_ref):
    d = jnp.sum(degs_ref[...], axis=0) + 1.0
    dinv = lax.rsqrt(d)
    o_ref[...] = (acc_ref[0, :N] + acc_ref[1, :N] + hn_ref[...]) * dinv[:, None]


def _comb_call(acc, hn, degs):
    return pl.pallas_call(
        _comb_body,
        out_shape=jax.ShapeDtypeStruct((N, D), jnp.float32),
    )(acc, hn, degs)


def kernel(x, edge_index_und, W, b):
    ei_flat = edge_index_und.reshape(2 * E)
    n_pad = GCHUNK * CHUNK - 2 * E
    src_p = jnp.concatenate([ei_flat, jnp.zeros((n_pad,), jnp.int32)])
    dst_p = jnp.concatenate([jnp.roll(ei_flat, -E),
                             jnp.full((n_pad,), DUMMY_DST, jnp.int32)])
    cidx = jnp.stack([src_p.reshape(GCHUNK, CHUNK),
                      dst_p.reshape(GCHUNK, CHUNK)], axis=1)
    degs = _deg_call(ei_flat)
    hn = _mm_call(x, W.T, b.reshape(1, D), degs)
    acc = _agg_call(jnp.concatenate([hn, hn], axis=1), cidx)
    return _comb_call(acc, hn, degs)


# speedup vs baseline: 1.1026x; 1.1026x over previous
import jax
import jax.numpy as jnp
from jax import lax
from jax.experimental import pallas as pl
from jax.experimental.pallas import tpu as pltpu
from jax.experimental.pallas import tpu_sc as plsc

N = 10000
E = 320000
D = 128
NC = 2
NS = 16
NW = NC * NS
L = 16

IPW = (2 * E) // NW

CHUNK = 128
NCHUNK = 160
GCHUNK = NW * NCHUNK
NPAD = 10240
DUMMY_DST = N + 100
ROWS_PER_TILE = NPAD // NS
EXP_CHUNK = 64
NEXP = ROWS_PER_TILE // EXP_CHUNK

def _mesh():
    return plsc.VectorSubcoreMesh(core_axis_name="c", subcore_axis_name="s")


def _deg_body(ei_hbm, degs_hbm, idx_v, hist_v):
    c = lax.axis_index("c")
    s = lax.axis_index("s")
    wid = c * NS + s

    zeros = jnp.zeros((L,), jnp.float32)

    def zero(i, carry):
        hist_v[pl.ds(i * L, L)] = zeros
        return carry

    lax.fori_loop(0, N // L, zero, 0)

    pltpu.sync_copy(ei_hbm.at[pl.ds(wid * IPW, IPW)], idx_v)

    ones = jnp.ones((L,), jnp.float32)

    def accum(i, carry):
        idx = idx_v[pl.ds(i * L, L)]
        plsc.addupdate_scatter(hist_v, [idx], ones)
        return carry

    lax.fori_loop(0, IPW // L, accum, 0)

    pltpu.sync_copy(hist_v, degs_hbm.at[wid])


def _deg_call(ei_flat):
    fn = pl.kernel(
        _deg_body,
        out_type=jax.ShapeDtypeStruct((NW, N), jnp.float32),
        mesh=_mesh(),
        scratch_types=[
            pltpu.VMEM((IPW,), jnp.int32),
            pltpu.VMEM((N,), jnp.float32),
        ],
        compiler_params=pltpu.CompilerParams(needs_layout_passes=False),
    )
    return fn(ei_flat)


def _mm_body(x_ref, wt_ref, b_ref, h_ref):
    h = jnp.dot(x_ref[...], wt_ref[...], preferred_element_type=jnp.float32)
    h_ref[...] = h + b_ref[...]


def _mm_call(x, wt, b2):
    return pl.pallas_call(
        _mm_body,
        out_shape=jax.ShapeDtypeStruct((N, D), jnp.float32),
    )(x, wt, b2)


def _scale_body(h_ref, degs_ref, hn_ref):
    d = jnp.sum(degs_ref[...], axis=0) + 1.0
    dinv = lax.rsqrt(d)
    hn_ref[...] = h_ref[...] * dinv[:, None]


def _scale_call(h, degs):
    return pl.pallas_call(
        _scale_body,
        out_shape=jax.ShapeDtypeStruct((N, D), jnp.float32),
    )(h, degs)


def _agg_body(hn_hbm, cidx_hbm, out_hbm,
              cs0, cs1, ds0, ds1, buf0, buf1, stage_v,
              acc_sh,
              isem0, isem1, gsem0, gsem1, ssem0, ssem1):
    c = lax.axis_index("c")
    s = lax.axis_index("s")
    wid = c * NS + s

    zeros = jnp.zeros((L,), jnp.float32)

    def zero(i, carry):
        r = lax.shift_right_logical(i, 3)
        col = lax.bitwise_and(i, 7)
        stage_v[r, pl.ds(col * L, L)] = zeros
        return carry

    lax.fori_loop(0, EXP_CHUNK * (D // L), zero, 0)

    for j in range(NEXP):
        base = s * ROWS_PER_TILE + j * EXP_CHUNK
        pltpu.sync_copy(stage_v, acc_sh.at[pl.ds(base, EXP_CHUNK)])
    plsc.subcore_barrier()

    slots = ((cs0, ds0, buf0, isem0, gsem0, ssem0),
             (cs1, ds1, buf1, isem1, gsem1, ssem1))

    def chunk_of(k):
        return k * NW + wid

    def copy_ds(p):
        cs, ds = slots[p][0], slots[p][1]
        for kk in range(CHUNK // L):
            ds[pl.ds(kk * L, L)] = cs[1, pl.ds(kk * L, L)]

    def start_idx(k, p):
        pltpu.async_copy(cidx_hbm.at[chunk_of(k)], slots[p][0], slots[p][3])

    def wait_idx(k, p):
        pltpu.make_async_copy(cidx_hbm.at[chunk_of(k)], slots[p][0],
                              slots[p][3]).wait()

    def start_gather(p):
        pltpu.async_copy(hn_hbm.at[slots[p][0].at[0]], slots[p][2],
                         slots[p][4])

    def wait_gather(p):
        pltpu.make_async_copy(hn_hbm.at[slots[p][0].at[0]], slots[p][2],
                              slots[p][4]).wait()

    def start_scatter(p):
        pltpu.async_copy(slots[p][2], acc_sh.at[slots[p][1]], slots[p][5],
                         add=True)

    def wait_scatter(p):
        pltpu.make_async_copy(slots[p][2], acc_sh.at[slots[p][1]],
                              slots[p][5]).wait()

    def step(k, p, do_idx=True, do_gather=True, do_wait_scatter=True):
        q = 1 - p
        if do_gather:
            wait_idx(k + 1, q)
            if do_wait_scatter:
                wait_scatter(q)
            start_gather(q)
        wait_gather(p)
        copy_ds(p)
        start_scatter(p)
        if do_idx:
            start_idx(k + 2, p)

    start_idx(jnp.int32(0), 0)
    wait_idx(jnp.int32(0), 0)
    start_gather(0)
    start_idx(jnp.int32(1), 1)

    step(jnp.int32(0), 0, do_wait_scatter=False)

    def pair(t, carry):
        step(2 * t + 1, 1)
        step(2 * t + 2, 0)
        return carry

    lax.fori_loop(0, (NCHUNK - 4) // 2, pair, 0)

    step(jnp.int32(NCHUNK - 3), 1)
    step(jnp.int32(NCHUNK - 2), 0, do_idx=False)
    step(jnp.int32(NCHUNK - 1), 1, do_idx=False, do_gather=False)
    wait_scatter(0)
    wait_scatter(1)

    plsc.subcore_barrier()

    for j in range(NEXP):
        base = s * ROWS_PER_TILE + j * EXP_CHUNK
        pltpu.sync_copy(acc_sh.at[pl.ds(base, EXP_CHUNK)],
                        out_hbm.at[c, pl.ds(base, EXP_CHUNK)])


def _agg_call(hn, cidx):
    fn = pl.kernel(
        _agg_body,
        out_type=jax.ShapeDtypeStruct((NC, NPAD, D), jnp.float32),
        mesh=_mesh(),
        scratch_types=[
            pltpu.VMEM((2, CHUNK), jnp.int32),
            pltpu.VMEM((2, CHUNK), jnp.int32),
            pltpu.VMEM((CHUNK,), jnp.int32),
            pltpu.VMEM((CHUNK,), jnp.int32),
            pltpu.VMEM((CHUNK, D), jnp.float32),
            pltpu.VMEM((CHUNK, D), jnp.float32),
            pltpu.VMEM((EXP_CHUNK, D), jnp.float32),
            pltpu.VMEM_SHARED((NPAD, D), jnp.float32),
            pltpu.SemaphoreType.DMA,
            pltpu.SemaphoreType.DMA,
            pltpu.SemaphoreType.DMA,
            pltpu.SemaphoreType.DMA,
            pltpu.SemaphoreType.DMA,
            pltpu.SemaphoreType.DMA,
        ],
        compiler_params=pltpu.CompilerParams(needs_layout_passes=False),
    )
    return fn(hn, cidx)


def _comb_body(acc_ref, hn_ref, degs_ref, o_ref):
    d = jnp.sum(degs_ref[...], axis=0) + 1.0
    dinv = lax.rsqrt(d)
    o_ref[...] = (acc_ref[0, :N] + acc_ref[1, :N] + hn_ref[...]) * dinv[:, None]


def _comb_call(acc, hn, degs):
    return pl.pallas_call(
        _comb_body,
        out_shape=jax.ShapeDtypeStruct((N, D), jnp.float32),
    )(acc, hn, degs)


def kernel(x, edge_index_und, W, b):
    ei_flat = edge_index_und.reshape(2 * E)
    n_pad = GCHUNK * CHUNK - 2 * E
    src_p = jnp.concatenate([ei_flat, jnp.zeros((n_pad,), jnp.int32)])
    dst_p = jnp.concatenate([jnp.roll(ei_flat, -E),
                             jnp.full((n_pad,), DUMMY_DST, jnp.int32)])
    cidx = jnp.stack([src_p.reshape(GCHUNK, CHUNK),
                      dst_p.reshape(GCHUNK, CHUNK)], axis=1)
    degs = _deg_call(ei_flat)
    h = _mm_call(x, W.T, b.reshape(1, D))
    hn = _scale_call(h, degs)
    acc = _agg_call(hn, cidx)
    return _comb_call(acc, hn, degs)


# speedup vs baseline: 1.1103x; 1.0069x over previous
import jax
import jax.numpy as jnp
from jax import lax
from jax.experimental import pallas as pl
from jax.experimental.pallas import tpu as pltpu
from jax.experimental.pallas import tpu_sc as plsc

N = 10000
E = 320000
D = 128
NC = 2
NS = 16
NW = NC * NS
L = 16

IPW = (2 * E) // NW

CHUNK = 128
NCHUNK = 160
GCHUNK = NW * NCHUNK
NPAD = 10240
DUMMY_DST = N + 100
ROWS_PER_TILE = NPAD // NS
EXP_CHUNK = 64
NEXP = ROWS_PER_TILE // EXP_CHUNK

def _mesh():
    return plsc.VectorSubcoreMesh(core_axis_name="c", subcore_axis_name="s")


def _deg_body(ei_hbm, degs_hbm, idx_v, hist_v):
    c = lax.axis_index("c")
    s = lax.axis_index("s")
    wid = c * NS + s

    zeros = jnp.zeros((L,), jnp.float32)

    def zero(i, carry):
        hist_v[pl.ds(i * L, L)] = zeros
        return carry

    lax.fori_loop(0, N // L, zero, 0)

    pltpu.sync_copy(ei_hbm.at[pl.ds(wid * IPW, IPW)], idx_v)

    ones = jnp.ones((L,), jnp.float32)

    def accum(i, carry):
        idx = idx_v[pl.ds(i * L, L)]
        plsc.addupdate_scatter(hist_v, [idx], ones)
        return carry

    lax.fori_loop(0, IPW // L, accum, 0)

    pltpu.sync_copy(hist_v, degs_hbm.at[wid])


def _deg_call(ei_flat):
    fn = pl.kernel(
        _deg_body,
        out_type=jax.ShapeDtypeStruct((NW, N), jnp.float32),
        mesh=_mesh(),
        scratch_types=[
            pltpu.VMEM((IPW,), jnp.int32),
            pltpu.VMEM((N,), jnp.float32),
        ],
        compiler_params=pltpu.CompilerParams(needs_layout_passes=False),
    )
    return fn(ei_flat)


def _mm_body(x_ref, wt_ref, b_ref, degs_ref, hn_ref):
    d = jnp.sum(degs_ref[...], axis=0) + 1.0
    dinv = lax.rsqrt(d)
    h = jnp.dot(x_ref[...], wt_ref[...], preferred_element_type=jnp.float32)
    hn_ref[...] = (h + b_ref[...]) * dinv[:, None]


def _mm_call(x, wt, b2, degs):
    return pl.pallas_call(
        _mm_body,
        out_shape=jax.ShapeDtypeStruct((N, D), jnp.float32),
    )(x, wt, b2, degs)


def _agg_body(hn_hbm, cidx_hbm, out_hbm,
              cs0, cs1, ds0, ds1, buf0, buf1, stage_v,
              acc_sh,
              isem0, isem1, gsem0, gsem1, ssem0, ssem1):
    c = lax.axis_index("c")
    s = lax.axis_index("s")
    wid = c * NS + s

    zeros = jnp.zeros((L,), jnp.float32)

    def zero(i, carry):
        r = lax.shift_right_logical(i, 3)
        col = lax.bitwise_and(i, 7)
        stage_v[r, pl.ds(col * L, L)] = zeros
        return carry

    lax.fori_loop(0, EXP_CHUNK * (D // L), zero, 0)

    for j in range(NEXP):
        base = s * ROWS_PER_TILE + j * EXP_CHUNK
        pltpu.sync_copy(stage_v, acc_sh.at[pl.ds(base, EXP_CHUNK)])
    plsc.subcore_barrier()

    slots = ((cs0, ds0, buf0, isem0, gsem0, ssem0),
             (cs1, ds1, buf1, isem1, gsem1, ssem1))

    def chunk_of(k):
        return k * NW + wid

    def copy_ds(p):
        cs, ds = slots[p][0], slots[p][1]
        for kk in range(CHUNK // L):
            ds[pl.ds(kk * L, L)] = cs[1, pl.ds(kk * L, L)]

    def start_idx(k, p):
        pltpu.async_copy(cidx_hbm.at[chunk_of(k)], slots[p][0], slots[p][3])

    def wait_idx(k, p):
        pltpu.make_async_copy(cidx_hbm.at[chunk_of(k)], slots[p][0],
                              slots[p][3]).wait()

    def start_gather(p):
        pltpu.async_copy(hn_hbm.at[slots[p][0].at[0]], slots[p][2],
                         slots[p][4])

    def wait_gather(p):
        pltpu.make_async_copy(hn_hbm.at[slots[p][0].at[0]], slots[p][2],
                              slots[p][4]).wait()

    def start_scatter(p):
        pltpu.async_copy(slots[p][2], acc_sh.at[slots[p][1]], slots[p][5],
                         add=True)

    def wait_scatter(p):
        pltpu.make_async_copy(slots[p][2], acc_sh.at[slots[p][1]],
                              slots[p][5]).wait()

    def step(k, p, do_idx=True, do_gather=True, do_wait_scatter=True):
        q = 1 - p
        if do_gather:
            wait_idx(k + 1, q)
            if do_wait_scatter:
                wait_scatter(q)
            start_gather(q)
        wait_gather(p)
        copy_ds(p)
        start_scatter(p)
        if do_idx:
            start_idx(k + 2, p)

    start_idx(jnp.int32(0), 0)
    wait_idx(jnp.int32(0), 0)
    start_gather(0)
    start_idx(jnp.int32(1), 1)

    step(jnp.int32(0), 0, do_wait_scatter=False)

    def pair(t, carry):
        step(2 * t + 1, 1)
        step(2 * t + 2, 0)
        return carry

    lax.fori_loop(0, (NCHUNK - 4) // 2, pair, 0)

    step(jnp.int32(NCHUNK - 3), 1)
    step(jnp.int32(NCHUNK - 2), 0, do_idx=False)
    step(jnp.int32(NCHUNK - 1), 1, do_idx=False, do_gather=False)
    wait_scatter(0)
    wait_scatter(1)

    plsc.subcore_barrier()

    for j in range(NEXP):
        base = s * ROWS_PER_TILE + j * EXP_CHUNK
        pltpu.sync_copy(acc_sh.at[pl.ds(base, EXP_CHUNK)],
                        out_hbm.at[c, pl.ds(base, EXP_CHUNK)])


def _agg_call(hn, cidx):
    fn = pl.kernel(
        _agg_body,
        out_type=jax.ShapeDtypeStruct((NC, NPAD, D), jnp.float32),
        mesh=_mesh(),
        scratch_types=[
            pltpu.VMEM((2, CHUNK), jnp.int32),
            pltpu.VMEM((2, CHUNK), jnp.int32),
            pltpu.VMEM((CHUNK,), jnp.int32),
            pltpu.VMEM((CHUNK,), jnp.int32),
            pltpu.VMEM((CHUNK, D), jnp.float32),
            pltpu.VMEM((CHUNK, D), jnp.float32),
            pltpu.VMEM((EXP_CHUNK, D), jnp.float32),
            pltpu.VMEM_SHARED((NPAD, D), jnp.float32),
            pltpu.SemaphoreType.DMA,
            pltpu.SemaphoreType.DMA,
            pltpu.SemaphoreType.DMA,
            pltpu.SemaphoreType.DMA,
            pltpu.SemaphoreType.DMA,
            pltpu.SemaphoreType.DMA,
        ],
        compiler_params=pltpu.CompilerParams(needs_layout_passes=False),
    )
    return fn(hn, cidx)


def _comb_body(acc_ref, hn_ref, degs_ref, o_ref):
    d = jnp.sum(degs_ref[...], axis=0) + 1.0
    dinv = lax.rsqrt(d)
    o_ref[...] = (acc_ref[0, :N] + acc_ref[1, :N] + hn_ref[...]) * dinv[:, None]


def _comb_call(acc, hn, degs):
    return pl.pallas_call(
        _comb_body,
        out_shape=jax.ShapeDtypeStruct((N, D), jnp.float32),
    )(acc, hn, degs)


def kernel(x, edge_index_und, W, b):
    ei_flat = edge_index_und.reshape(2 * E)
    n_pad = GCHUNK * CHUNK - 2 * E
    src_p = jnp.concatenate([ei_flat, jnp.zeros((n_pad,), jnp.int32)])
    dst_p = jnp.concatenate([jnp.roll(ei_flat, -E),
                             jnp.full((n_pad,), DUMMY_DST, jnp.int32)])
    cidx = jnp.stack([src_p.reshape(GCHUNK, CHUNK),
                      dst_p.reshape(GCHUNK, CHUNK)], axis=1)
    degs = _deg_call(ei_flat)
    hn = _mm_call(x, W.T, b.reshape(1, D), degs)
    acc = _agg_call(hn, cidx)
    return _comb_call(acc, hn, degs)


# speedup vs baseline: 1.7730x; 1.5969x over previous
import jax
import jax.numpy as jnp
from jax import lax
from jax.experimental import pallas as pl
from jax.experimental.pallas import tpu as pltpu
from jax.experimental.pallas import tpu_sc as plsc

N = 10000
E = 320000
D = 128
NC = 2
NS = 16
NW = NC * NS
L = 16

IPW = (2 * E) // NW

CHUNK = 128
NCHUNK = 158
GCHUNK = NW * NCHUNK
NPAD = 10240
DUMMY_DST = N + 100
ROWS_PER_TILE = NPAD // NS
EXP_CHUNK = 64
NEXP = ROWS_PER_TILE // EXP_CHUNK

def _mesh():
    return plsc.VectorSubcoreMesh(core_axis_name="c", subcore_axis_name="s")


def _deg_body(ei_hbm, degs_hbm, idx_v, hist_v):
    c = lax.axis_index("c")
    s = lax.axis_index("s")
    wid = c * NS + s

    zeros = jnp.zeros((L,), jnp.float32)

    def zero(i, carry):
        hist_v[pl.ds(i * L, L)] = zeros
        return carry

    lax.fori_loop(0, N // L, zero, 0)

    pltpu.sync_copy(ei_hbm.at[pl.ds(wid * IPW, IPW)], idx_v)

    ones = jnp.ones((L,), jnp.float32)

    def accum(i, carry):
        idx = idx_v[pl.ds(i * L, L)]
        plsc.addupdate_scatter(hist_v, [idx], ones)
        return carry

    lax.fori_loop(0, IPW // L, accum, 0)

    pltpu.sync_copy(hist_v, degs_hbm.at[wid])


def _deg_call(ei_flat):
    fn = pl.kernel(
        _deg_body,
        out_type=jax.ShapeDtypeStruct((NW, N), jnp.float32),
        mesh=_mesh(),
        scratch_types=[
            pltpu.VMEM((IPW,), jnp.int32),
            pltpu.VMEM((N,), jnp.float32),
        ],
        compiler_params=pltpu.CompilerParams(needs_layout_passes=False),
    )
    return fn(ei_flat)


def _mm_body(x_ref, wt_ref, b_ref, degs_ref, hn_ref):
    d = jnp.sum(degs_ref[...], axis=0) + 1.0
    dinv = lax.rsqrt(d)
    h = jnp.dot(x_ref[...], wt_ref[...], preferred_element_type=jnp.float32)
    hn_ref[...] = (h + b_ref[...]) * dinv[:, None]


def _mm_call(x, wt, b2, degs):
    return pl.pallas_call(
        _mm_body,
        out_shape=jax.ShapeDtypeStruct((N, D), jnp.float32),
    )(x, wt, b2, degs)


def _agg_body(hn_hbm, cidx_hbm, out_hbm,
              cs0, cs1, ds0, ds1, buf0, buf1, stage_v,
              acc_sh,
              isem0, isem1, gsem0, gsem1, ssem0, ssem1, esem):
    c = lax.axis_index("c")
    s = lax.axis_index("s")
    wid = c * NS + s

    zeros = jnp.zeros((L,), jnp.float32)

    def zero(i, carry):
        r = lax.shift_right_logical(i, 3)
        col = lax.bitwise_and(i, 7)
        stage_v[r, pl.ds(col * L, L)] = zeros
        return carry

    lax.fori_loop(0, EXP_CHUNK * (D // L), zero, 0)

    for j in range(NEXP):
        base = s * ROWS_PER_TILE + j * EXP_CHUNK
        pltpu.async_copy(stage_v, acc_sh.at[pl.ds(base, EXP_CHUNK)], esem)
    for j in range(NEXP):
        base = s * ROWS_PER_TILE + j * EXP_CHUNK
        pltpu.make_async_copy(stage_v, acc_sh.at[pl.ds(base, EXP_CHUNK)],
                              esem).wait()
    plsc.subcore_barrier()

    slots = ((cs0, ds0, buf0, isem0, gsem0, ssem0),
             (cs1, ds1, buf1, isem1, gsem1, ssem1))

    def chunk_of(k):
        return k * NW + wid

    def copy_ds(p):
        cs, ds = slots[p][0], slots[p][1]
        for kk in range(CHUNK // L):
            ds[pl.ds(kk * L, L)] = cs[1, pl.ds(kk * L, L)]

    def start_idx(k, p):
        pltpu.async_copy(cidx_hbm.at[chunk_of(k)], slots[p][0], slots[p][3])

    def wait_idx(k, p):
        pltpu.make_async_copy(cidx_hbm.at[chunk_of(k)], slots[p][0],
                              slots[p][3]).wait()

    def start_gather(p):
        pltpu.async_copy(hn_hbm.at[slots[p][0].at[0]], slots[p][2],
                         slots[p][4])

    def wait_gather(p):
        pltpu.make_async_copy(hn_hbm.at[slots[p][0].at[0]], slots[p][2],
                              slots[p][4]).wait()

    def start_scatter(p):
        pltpu.async_copy(slots[p][2], acc_sh.at[slots[p][1]], slots[p][5],
                         add=True)

    def wait_scatter(p):
        pltpu.make_async_copy(slots[p][2], acc_sh.at[slots[p][1]],
                              slots[p][5]).wait()

    def step(k, p, do_idx=True, do_gather=True, do_wait_scatter=True):
        q = 1 - p
        if do_gather:
            wait_idx(k + 1, q)
            if do_wait_scatter:
                wait_scatter(q)
            start_gather(q)
        wait_gather(p)
        copy_ds(p)
        start_scatter(p)
        if do_idx:
            start_idx(k + 2, p)

    start_idx(jnp.int32(0), 0)
    wait_idx(jnp.int32(0), 0)
    start_gather(0)
    start_idx(jnp.int32(1), 1)

    step(jnp.int32(0), 0, do_wait_scatter=False)

    def pair(t, carry):
        step(2 * t + 1, 1)
        step(2 * t + 2, 0)
        return carry

    lax.fori_loop(0, (NCHUNK - 4) // 2, pair, 0)

    step(jnp.int32(NCHUNK - 3), 1)
    step(jnp.int32(NCHUNK - 2), 0, do_idx=False)
    step(jnp.int32(NCHUNK - 1), 1, do_idx=False, do_gather=False)
    wait_scatter(0)
    wait_scatter(1)

    plsc.subcore_barrier()

    for j in range(NEXP):
        base = s * ROWS_PER_TILE + j * EXP_CHUNK
        pltpu.async_copy(acc_sh.at[pl.ds(base, EXP_CHUNK)],
                         out_hbm.at[c, pl.ds(base, EXP_CHUNK)], esem)
    for j in range(NEXP):
        base = s * ROWS_PER_TILE + j * EXP_CHUNK
        pltpu.make_async_copy(acc_sh.at[pl.ds(base, EXP_CHUNK)],
                              out_hbm.at[c, pl.ds(base, EXP_CHUNK)],
                              esem).wait()


def _agg_call(hn, cidx):
    fn = pl.kernel(
        _agg_body,
        out_type=jax.ShapeDtypeStruct((NC, NPAD, D), jnp.float32),
        mesh=_mesh(),
        scratch_types=[
            pltpu.VMEM((2, CHUNK), jnp.int32),
            pltpu.VMEM((2, CHUNK), jnp.int32),
            pltpu.VMEM((CHUNK,), jnp.int32),
            pltpu.VMEM((CHUNK,), jnp.int32),
            pltpu.VMEM((CHUNK, D), jnp.float32),
            pltpu.VMEM((CHUNK, D), jnp.float32),
            pltpu.VMEM((EXP_CHUNK, D), jnp.float32),
            pltpu.VMEM_SHARED((NPAD, D), jnp.float32),
            pltpu.SemaphoreType.DMA,
            pltpu.SemaphoreType.DMA,
            pltpu.SemaphoreType.DMA,
            pltpu.SemaphoreType.DMA,
            pltpu.SemaphoreType.DMA,
            pltpu.SemaphoreType.DMA,
            pltpu.SemaphoreType.DMA,
        ],
        compiler_params=pltpu.CompilerParams(needs_layout_passes=False),
    )
    return fn(hn, cidx)


def _comb_body(acc_ref, hn_ref, degs_ref, o_ref):
    d = jnp.sum(degs_ref[...], axis=0) + 1.0
    dinv = lax.rsqrt(d)
    o_ref[...] = (acc_ref[0, :N] + acc_ref[1, :N] + hn_ref[...]) * dinv[:, None]


def _comb_call(acc, hn, degs):
    return pl.pallas_call(
        _comb_body,
        out_shape=jax.ShapeDtypeStruct((N, D), jnp.float32),
    )(acc, hn, degs)


def kernel(x, edge_index_und, W, b):
    ei_flat = edge_index_und.reshape(2 * E)
    n_pad = GCHUNK * CHUNK - 2 * E
    src_p = jnp.concatenate([ei_flat, jnp.zeros((n_pad,), jnp.int32)])
    dst_p = jnp.concatenate([jnp.roll(ei_flat, -E),
                             jnp.full((n_pad,), DUMMY_DST, jnp.int32)])
    cidx = jnp.stack([src_p.reshape(GCHUNK, CHUNK),
                      dst_p.reshape(GCHUNK, CHUNK)], axis=1)
    degs = _deg_call(ei_flat)
    hn = _mm_call(x, W.T, b.reshape(1, D), degs)
    acc = _agg_call(hn, cidx)
    return _comb_call(acc, hn, degs)
